# baseline (speedup 1.0000x reference)
import jax
import jax.numpy as jnp
from jax.experimental import pallas as pl
from jax.experimental.pallas import tpu as pltpu

N = 4096
D = 256
H1 = 64
H2 = 32
K = 16
HC = 2 * H2 + K

BM = 256
NB = N // BM
BR = 512
NRB = N // BR


def _gcn_body(adj_ref, x_ref, w1_ref, wcat_ref,
              packed_ref,
              abf_ref, p_ref, h1_ref, q_ref):
    s = pl.program_id(0)
    j = pl.program_id(1)
    rows = pl.ds(j * BM, BM)

    @pl.when(s == 0)
    def _():
        p_ref[rows, :] = jnp.dot(
            x_ref[...], w1_ref[...],
            preferred_element_type=jnp.float32).astype(jnp.bfloat16)

    @pl.when(s == 1)
    def _():
        ablk = adj_ref[...].astype(jnp.bfloat16)
        abf_ref[rows, :] = ablk
        h1_ref[rows, :] = jnp.maximum(
            jnp.dot(ablk, p_ref[...],
                    preferred_element_type=jnp.float32), 0.0
        ).astype(jnp.bfloat16)

    @pl.when((s == 2) & (j == 0))
    def _():
        q_ref[...] = jnp.dot(h1_ref[...], wcat_ref[...],
                             preferred_element_type=jnp.float32
                             ).astype(jnp.bfloat16)

    @pl.when(s == 2)
    def _():
        out2 = jnp.dot(abf_ref[rows, :], q_ref[...],
                       preferred_element_type=jnp.float32)
        semi = out2[:, 2 * H2:]
        packed_ref[rows, :HC] = out2
        m = jnp.max(semi, axis=1, keepdims=True)
        shifted = semi - m
        e = jnp.exp(shifted)
        ssum = jnp.sum(e, axis=1, keepdims=True)
        packed_ref[rows, HC:HC + K] = shifted - jnp.log(ssum)
        packed_ref[rows, HC + K:HC + 2 * K] = e / ssum
        packed_ref[rows, HC + 2 * K:] = jnp.zeros((BM, 128 - HC - 2 * K),
                                                  jnp.float32)


def _recon_body(zb_ref, zall_ref, out_ref):
    out_ref[...] = jax.lax.dot_general(
        zb_ref[...], zall_ref[...],
        dimension_numbers=(((1,), (1,)), ((), ())),
        preferred_element_type=jnp.float32)


@jax.jit
def kernel(x, adj, W1, W2, W3, Wsemi):
    wcat = jnp.concatenate([W2, W3, Wsemi], axis=1).astype(jnp.bfloat16)

    def adj_map(s, j):
        return (jnp.where(s < 1, 0, jnp.where(s > 1, NB - 1, j)), 0)

    def x_map(s, j):
        return (jnp.where(s < 1, j, NB - 1), 0)

    packed = pl.pallas_call(
        _gcn_body,
        grid=(3, NB),
        in_specs=[
            pl.BlockSpec((BM, N), adj_map),
            pl.BlockSpec((BM, D), x_map),
            pl.BlockSpec((D, H1), lambda s, j: (0, 0)),
            pl.BlockSpec((H1, HC), lambda s, j: (0, 0)),
        ],
        out_specs=pl.BlockSpec((N, 128), lambda s, j: (0, 0)),
        out_shape=jax.ShapeDtypeStruct((N, 128), jnp.float32),
        scratch_shapes=[
            pltpu.VMEM((N, N), jnp.bfloat16),
            pltpu.VMEM((N, H1), jnp.bfloat16),
            pltpu.VMEM((N, H1), jnp.bfloat16),
            pltpu.VMEM((N, HC), jnp.bfloat16),
        ],
        compiler_params=pltpu.CompilerParams(
            dimension_semantics=("arbitrary", "arbitrary")),
    )(adj, x, W1, wcat)

    mean = packed[:, :H2]
    logvar = packed[:, H2:2 * H2]
    semi = packed[:, 2 * H2:HC]
    logsm = packed[:, HC:HC + K]
    sm = packed[:, HC + K:HC + 2 * K]
    z = mean

    adj_recon = pl.pallas_call(
        _recon_body,
        grid=(NRB,),
        in_specs=[
            pl.BlockSpec((BR, H2), lambda j: (j, 0)),
            pl.BlockSpec((N, H2), lambda j: (0, 0)),
        ],
        out_specs=pl.BlockSpec((BR, N), lambda j: (j, 0)),
        out_shape=jax.ShapeDtypeStruct((N, N), jnp.float32),
        compiler_params=pltpu.CompilerParams(
            dimension_semantics=("arbitrary",)),
    )(mean, mean)

    return (adj_recon, mean, logvar, z, logsm, sm, semi)


# speedup vs baseline: 1.0825x; 1.0825x over previous
import jax
import jax.numpy as jnp
from jax.experimental import pallas as pl
from jax.experimental.pallas import tpu as pltpu

N = 4096
D = 256
H1 = 64
H2 = 32
K = 16
HC = 2 * H2 + K

BM = 256
NB = N // BM


def _body(adj_ref, x_ref, w1_ref, wcat_ref,
          packed_ref, recon_ref,
          abf_ref, p_ref, qf_ref, acc_ref):
    s = pl.program_id(0)
    j = pl.program_id(1)
    rows = pl.ds(j * BM, BM)

    @pl.when(s == 0)
    def _():
        p_ref[rows, :] = jnp.dot(
            x_ref[...], w1_ref[...],
            preferred_element_type=jnp.float32).astype(jnp.bfloat16)
        abf_ref[rows, :] = jnp.zeros((BM, N), jnp.bfloat16)

    @pl.when((s == 0) & (j == 0))
    def _():
        qf_ref[...] = jnp.zeros((N, HC), jnp.bfloat16)
        acc_ref[...] = jnp.zeros((N, HC), jnp.float32)

    @pl.when(s == 1)
    def _():
        ablk = adj_ref[...].astype(jnp.bfloat16)
        abf_ref[rows, :] = ablk
        h1 = jnp.maximum(
            jnp.dot(ablk, p_ref[...],
                    preferred_element_type=jnp.float32), 0.0
        ).astype(jnp.bfloat16)
        qk = jnp.dot(h1, wcat_ref[...],
                     preferred_element_type=jnp.float32).astype(jnp.bfloat16)
        acc_ref[rows, :] += jnp.dot(ablk, qf_ref[...],
                                    preferred_element_type=jnp.float32)
        qf_ref[rows, :] = qk
        acc_ref[...] += jnp.dot(abf_ref[:, rows], qk,
                                preferred_element_type=jnp.float32)

    @pl.when(s == 2)
    def _():
        out2 = acc_ref[rows, :]
        semi = out2[:, 2 * H2:]
        packed_ref[rows, :HC] = out2
        m = jnp.max(semi, axis=1, keepdims=True)
        shifted = semi - m
        e = jnp.exp(shifted)
        ssum = jnp.sum(e, axis=1, keepdims=True)
        packed_ref[rows, HC:HC + K] = shifted - jnp.log(ssum)
        packed_ref[rows, HC + K:HC + 2 * K] = e / ssum
        packed_ref[rows, HC + 2 * K:] = jnp.zeros((BM, 128 - HC - 2 * K),
                                                  jnp.float32)
        recon_ref[...] = jax.lax.dot_general(
            out2[:, :H2], acc_ref[:, :H2],
            dimension_numbers=(((1,), (1,)), ((), ())),
            preferred_element_type=jnp.float32)


@jax.jit
def kernel(x, adj, W1, W2, W3, Wsemi):
    wcat = jnp.concatenate([W2, W3, Wsemi], axis=1).astype(jnp.bfloat16)

    def adj_map(s, j):
        return (jnp.where(s < 1, 0, jnp.where(s > 1, NB - 1, j)), 0)

    def x_map(s, j):
        return (jnp.where(s < 1, j, NB - 1), 0)

    def recon_map(s, j):
        return (jnp.where(s < 2, 0, j), 0)

    packed, adj_recon = pl.pallas_call(
        _body,
        grid=(3, NB),
        in_specs=[
            pl.BlockSpec((BM, N), adj_map),
            pl.BlockSpec((BM, D), x_map),
            pl.BlockSpec((D, H1), lambda s, j: (0, 0)),
            pl.BlockSpec((H1, HC), lambda s, j: (0, 0)),
        ],
        out_specs=[
            pl.BlockSpec((N, 128), lambda s, j: (0, 0)),
            pl.BlockSpec((BM, N), recon_map),
        ],
        out_shape=[
            jax.ShapeDtypeStruct((N, 128), jnp.float32),
            jax.ShapeDtypeStruct((N, N), jnp.float32),
        ],
        scratch_shapes=[
            pltpu.VMEM((N, N), jnp.bfloat16),
            pltpu.VMEM((N, H1), jnp.bfloat16),
            pltpu.VMEM((N, HC), jnp.bfloat16),
            pltpu.VMEM((N, HC), jnp.float32),
        ],
        compiler_params=pltpu.CompilerParams(
            dimension_semantics=("arbitrary", "arbitrary")),
    )(adj, x, W1, wcat)

    mean = packed[:, :H2]
    logvar = packed[:, H2:2 * H2]
    semi = packed[:, 2 * H2:HC]
    logsm = packed[:, HC:HC + K]
    sm = packed[:, HC + K:HC + 2 * K]
    z = mean

    return (adj_recon, mean, logvar, z, logsm, sm, semi)


# speedup vs baseline: 1.1388x; 1.0520x over previous
import jax
import jax.numpy as jnp
from jax.experimental import pallas as pl
from jax.experimental.pallas import tpu as pltpu

N = 4096
D = 256
H1 = 64
H2 = 32
K = 16
HC = 2 * H2 + K

BM = 256
NB = N // BM


def _body(adj_ref, x_ref, w1_ref, wcat_ref,
          mean_ref, logvar_ref, z_ref, semi_ref, logsm_ref, sm_ref,
          recon_ref,
          abf_ref, p_ref, qf_ref, acc_ref):
    s = pl.program_id(0)
    j = pl.program_id(1)
    rows = pl.ds(j * BM, BM)

    @pl.when(s == 0)
    def _():
        p_ref[rows, :] = jnp.dot(
            x_ref[...], w1_ref[...],
            preferred_element_type=jnp.float32).astype(jnp.bfloat16)
        abf_ref[rows, :] = jnp.zeros((BM, N), jnp.bfloat16)

    @pl.when((s == 0) & (j == 0))
    def _():
        qf_ref[...] = jnp.zeros((N, HC), jnp.bfloat16)
        acc_ref[...] = jnp.zeros((N, HC), jnp.float32)

    @pl.when(s == 1)
    def _():
        ablk = adj_ref[...].astype(jnp.bfloat16)
        abf_ref[rows, :] = ablk
        h1 = jnp.maximum(
            jnp.dot(ablk, p_ref[...],
                    preferred_element_type=jnp.float32), 0.0
        ).astype(jnp.bfloat16)
        qk = jnp.dot(h1, wcat_ref[...],
                     preferred_element_type=jnp.float32).astype(jnp.bfloat16)
        acc_ref[rows, :] += jnp.dot(ablk, qf_ref[...],
                                    preferred_element_type=jnp.float32)
        qf_ref[rows, :] = qk
        acc_ref[...] += jnp.dot(abf_ref[:, rows], qk,
                                preferred_element_type=jnp.float32)

    @pl.when(s == 2)
    def _():
        out2 = acc_ref[rows, :]
        mean = out2[:, :H2]
        logvar = out2[:, H2:2 * H2]
        semi = out2[:, 2 * H2:]
        mean_ref[...] = mean
        z_ref[...] = mean
        logvar_ref[...] = logvar
        semi_ref[...] = semi
        m = jnp.max(semi, axis=1, keepdims=True)
        shifted = semi - m
        e = jnp.exp(shifted)
        ssum = jnp.sum(e, axis=1, keepdims=True)
        sm_ref[...] = e / ssum
        logsm_ref[...] = shifted - jnp.log(ssum)
        recon_ref[...] = jax.lax.dot_general(
            mean, acc_ref[:, :H2],
            dimension_numbers=(((1,), (1,)), ((), ())),
            preferred_element_type=jnp.float32)


@jax.jit
def kernel(x, adj, W1, W2, W3, Wsemi):
    wcat = jnp.concatenate([W2, W3, Wsemi], axis=1).astype(jnp.bfloat16)

    def adj_map(s, j):
        return (jnp.where(s < 1, 0, jnp.where(s > 1, NB - 1, j)), 0)

    def x_map(s, j):
        return (jnp.where(s < 1, j, NB - 1), 0)

    def out_map(s, j):
        return (jnp.where(s < 2, 0, j), 0)

    mean, logvar, z, semi, logsm, sm, adj_recon = pl.pallas_call(
        _body,
        grid=(3, NB),
        in_specs=[
            pl.BlockSpec((BM, N), adj_map),
            pl.BlockSpec((BM, D), x_map),
            pl.BlockSpec((D, H1), lambda s, j: (0, 0)),
            pl.BlockSpec((H1, HC), lambda s, j: (0, 0)),
        ],
        out_specs=[
            pl.BlockSpec((BM, H2), out_map),
            pl.BlockSpec((BM, H2), out_map),
            pl.BlockSpec((BM, H2), out_map),
            pl.BlockSpec((BM, K), out_map),
            pl.BlockSpec((BM, K), out_map),
            pl.BlockSpec((BM, K), out_map),
            pl.BlockSpec((BM, N), out_map),
        ],
        out_shape=[
            jax.ShapeDtypeStruct((N, H2), jnp.float32),
            jax.ShapeDtypeStruct((N, H2), jnp.float32),
            jax.ShapeDtypeStruct((N, H2), jnp.float32),
            jax.ShapeDtypeStruct((N, K), jnp.float32),
            jax.ShapeDtypeStruct((N, K), jnp.float32),
            jax.ShapeDtypeStruct((N, K), jnp.float32),
            jax.ShapeDtypeStruct((N, N), jnp.float32),
        ],
        scratch_shapes=[
            pltpu.VMEM((N, N), jnp.bfloat16),
            pltpu.VMEM((N, H1), jnp.bfloat16),
            pltpu.VMEM((N, HC), jnp.bfloat16),
            pltpu.VMEM((N, HC), jnp.float32),
        ],
        compiler_params=pltpu.CompilerParams(
            dimension_semantics=("arbitrary", "arbitrary")),
    )(adj, x, W1, wcat)

    return (adj_recon, mean, logvar, z, logsm, sm, semi)
